# baseline (speedup 1.0000x reference)
import functools

import jax
import jax.numpy as jnp
from jax import lax
from jax.experimental import pallas as pl
from jax.experimental.pallas import tpu as pltpu
from jax.experimental.pallas import tpu_sc as plsc

_H = 128
_K = 8
_THR = 1e-06
_X = 8192
_L = 16
_RPW = 4


_NCH = 8
_CW = _X // _NCH


def _front_sc(x_hbm, out_hbm,
              d0_r, d1_r, d2_r, d3_r, c0_r, c1_r, c2_r, c3_r,
              st, *sems):
    wid = lax.axis_index("s") * 2 + lax.axis_index("c")
    base = wid * _RPW
    dbufs = (d0_r, d1_r, d2_r, d3_r)
    cbufs = (c0_r, c1_r, c2_r, c3_r)
    copies = [[] for _ in range(_NCH)]
    for ch in range(_NCH):
        for r in range(_RPW):
            copies[ch].append(pltpu.async_copy(
                x_hbm.at[base + r, 0, pl.ds(ch * _CW, _CW)],
                dbufs[r].at[pl.ds(ch * _CW, _CW)], sems[ch]))
            copies[ch].append(pltpu.async_copy(
                x_hbm.at[base + r, 1, pl.ds(ch * _CW, _CW)],
                cbufs[r].at[pl.ds(ch * _CW, _CW)], sems[ch]))

    inf = jnp.float32(jnp.inf)
    iota = lax.iota(jnp.int32, _L)
    lo8 = iota < _K

    def body(j, carry):
        bests, bidxs, cnts = carry
        nb, ni, nc = [], [], []
        off = j * _L
        for r in range(_RPW):
            d0 = dbufs[r][pl.ds(off, _L)]
            d1 = dbufs[r][pl.ds(off + 1, _L)]
            c0 = cbufs[r][pl.ds(off, _L)]
            c1 = cbufs[r][pl.ds(off + 1, _L)]
            gidx = off + iota
            disc = (jnp.abs(d0 - d1) > _THR) & (gidx < _X - 1)
            score = jnp.where(disc, (c0 + c1) * 0.5, inf)
            scand, sidx = plsc.sort_key_val(score, gidx)
            mk = jnp.where(lo8, bests[r], lax.rev(scand, (0,)))
            mi = jnp.where(lo8, bidxs[r], lax.rev(sidx, (0,)))
            b2, i2 = plsc.sort_key_val(mk, mi)
            nb.append(b2)
            ni.append(i2)
            nc.append(cnts[r] + disc.astype(jnp.int32))
        return (tuple(nb), tuple(ni), tuple(nc))

    carry = (tuple(jnp.full((_L,), inf) for _ in range(_RPW)),
             tuple(jnp.zeros((_L,), jnp.int32) for _ in range(_RPW)),
             tuple(jnp.zeros((_L,), jnp.int32) for _ in range(_RPW)))
    spc = _CW // _L
    for cp in copies[0]:
        cp.wait()
    for ch in range(_NCH):
        if ch + 1 < _NCH:
            for cp in copies[ch + 1]:
                cp.wait()
        carry = lax.fori_loop(ch * spc, (ch + 1) * spc, body, carry)
    bests, bidxs, cnts = carry

    for r in range(_RPW):
        bv, bi, cv = bests[r], bidxs[r], cnts[r]
        uLg = plsc.load_gather(dbufs[r], [bi])
        uRg = plsc.load_gather(dbufs[r], [bi + 1])
        cLg = plsc.load_gather(cbufs[r], [bi])
        cRg = plsc.load_gather(cbufs[r], [bi + 1])
        st[r, pl.ds(0, _L)] = uLg
        st[r, pl.ds(_L, _L)] = uRg
        st[r, pl.ds(2 * _L, _L)] = (cLg + cRg) * 0.5
        st[r, pl.ds(3 * _L, _L)] = jnp.where(bv < inf, jnp.ones((_L,), jnp.float32),
                                             jnp.zeros((_L,), jnp.float32))
        st[r, pl.ds(4 * _L, _L)] = lax.broadcast(jnp.sum(cv).astype(jnp.float32), (_L,))

    pltpu.sync_copy(st, out_hbm.at[pl.ds(base, _RPW)])


def _ln(h, g, b):
    mu = jnp.mean(h, axis=-1, keepdims=True)
    var = jnp.mean((h - mu) ** 2, axis=-1, keepdims=True)
    return (h - mu) / jnp.sqrt(var + 1e-5) * g + b


def _gelu(h):
    return 0.5 * h * (1.0 + jax.lax.erf(h * 0.7071067811865476))


def _mlp_kernel(fr_ref, Win_ref, bin_ref, gin_ref, bein_ref,
                g1_ref, be1_ref, W1a_ref, b1a_ref, W1b_ref, b1b_ref,
                g2_ref, be2_ref, W2a_ref, b2a_ref, W2b_ref, b2b_ref,
                g3_ref, be3_ref, W3a_ref, b3a_ref, W3b_ref, b3b_ref,
                gh_ref, beh_ref, Wh1_ref, bh1_ref, Wh2_ref, bh2_ref,
                o_ref):
    uL = fr_ref[:, 0:_K]
    uR = fr_ref[:, _L:_L + _K]
    diff = uL - uR
    feats = (uL, uR, diff, jnp.abs(diff), (uL + uR) * 0.5, jnp.sign(diff))
    Win = Win_ref[...]
    h3 = feats[0][:, :, None] * Win[0][None, None, :]
    for f in range(1, 6):
        h3 = h3 + feats[f][:, :, None] * Win[f][None, None, :]
    bb, kk, hh = h3.shape
    h = h3.reshape(bb * kk, hh) + bin_ref[...]
    h = _gelu(_ln(h, gin_ref[...], bein_ref[...]))
    for (g_r, be_r, Wa_r, ba_r, Wb_r, bb_r) in (
            (g1_ref, be1_ref, W1a_ref, b1a_ref, W1b_ref, b1b_ref),
            (g2_ref, be2_ref, W2a_ref, b2a_ref, W2b_ref, b2b_ref),
            (g3_ref, be3_ref, W3a_ref, b3a_ref, W3b_ref, b3b_ref)):
        r = _ln(h, g_r[...], be_r[...])
        r = _gelu(jnp.dot(r, Wa_r[...], preferred_element_type=jnp.float32) + ba_r[...])
        r = jnp.dot(r, Wb_r[...], preferred_element_type=jnp.float32) + bb_r[...]
        h = h + r
    o = _ln(h, gh_ref[...], beh_ref[...])
    o = _gelu(jnp.dot(o, Wh1_ref[...], preferred_element_type=jnp.float32) + bh1_ref[...])
    o = jnp.dot(o, Wh2_ref[...], preferred_element_type=jnp.float32) + bh2_ref[...]
    o_ref[...] = o


def kernel(x, W_in, b_in, g_in, be_in, g1, be1, W1a, b1a, W1b, b1b,
           g2, be2, W2a, b2a, W2b, b2b, g3, be3, W3a, b3a, W3b, b3b,
           g_h, be_h, Wh1, bh1, Wh2, bh2):
    B, C, X = x.shape
    f32 = jnp.float32

    front = pl.kernel(
        _front_sc,
        out_type=jax.ShapeDtypeStruct((B, 5 * _L), f32),
        mesh=plsc.VectorSubcoreMesh(core_axis_name="c", subcore_axis_name="s",
                                    num_cores=2, num_subcores=16),
        compiler_params=pltpu.CompilerParams(needs_layout_passes=False),
        scratch_types=[pltpu.VMEM((_X + _L,), f32)] * 8
        + [pltpu.VMEM((_RPW, 5 * _L), f32)]
        + [pltpu.SemaphoreType.DMA] * _NCH,
    )
    fr = front(x)

    Wh1p = jnp.pad(Wh1, ((0, 0), (0, _H - Wh1.shape[1])))
    bh1p = jnp.pad(bh1, (0, _H - bh1.shape[0]))
    Wh2p = jnp.pad(Wh2, ((0, _H - Wh2.shape[0]), (0, _H - Wh2.shape[1])))
    bh2p = jnp.pad(bh2, (0, _H - bh2.shape[0]))
    row = lambda v: v.reshape(1, -1)
    o_pad = pl.pallas_call(
        _mlp_kernel,
        out_shape=jax.ShapeDtypeStruct((B * _K, _H), f32),
    )(fr, W_in, row(b_in), row(g_in), row(be_in),
      row(g1), row(be1), W1a, row(b1a), W1b, row(b1b),
      row(g2), row(be2), W2a, row(b2a), W2b, row(b2b),
      row(g3), row(be3), W3a, row(b3a), W3b, row(b3b),
      row(g_h), row(be_h), Wh1p, row(bh1p), Wh2p, row(bh2p))

    fp_mlp = o_pad[:, :2].reshape(B, _K, 2).transpose(0, 2, 1)
    fr4 = jnp.stack([fr[:, p * _L:p * _L + _K] for p in range(4)], axis=1)
    fp = jnp.concatenate([fp_mlp, fr4], axis=1)
    front_count = fr[:, 4 * _L].astype(jnp.int32)
    return (fp, front_count)


# speedup vs baseline: 1.0201x; 1.0201x over previous
import functools

import jax
import jax.numpy as jnp
from jax import lax
from jax.experimental import pallas as pl
from jax.experimental.pallas import tpu as pltpu
from jax.experimental.pallas import tpu_sc as plsc

_H = 128
_K = 8
_THR = 1e-06
_X = 8192
_L = 16
_RPW = 4


_NCH = 4
_CW = _X // _NCH


def _front_sc(x_hbm, out_hbm,
              d0_r, d1_r, d2_r, d3_r, c0_r, c1_r, c2_r, c3_r,
              st, *sems):
    wid = lax.axis_index("s") * 2 + lax.axis_index("c")
    base = wid * _RPW
    dbufs = (d0_r, d1_r, d2_r, d3_r)
    cbufs = (c0_r, c1_r, c2_r, c3_r)
    copies = [[] for _ in range(_NCH)]
    for ch in range(_NCH):
        for r in range(_RPW):
            copies[ch].append(pltpu.async_copy(
                x_hbm.at[base + r, 0, pl.ds(ch * _CW, _CW)],
                dbufs[r].at[pl.ds(ch * _CW, _CW)], sems[ch]))
            copies[ch].append(pltpu.async_copy(
                x_hbm.at[base + r, 1, pl.ds(ch * _CW, _CW)],
                cbufs[r].at[pl.ds(ch * _CW, _CW)], sems[ch]))

    inf = jnp.float32(jnp.inf)
    iota = lax.iota(jnp.int32, _L)
    lo8 = iota < _K

    def body(j, carry):
        bests, bidxs, cnts = carry
        nb, ni, nc = [], [], []
        off = j * _L
        for r in range(_RPW):
            d0 = dbufs[r][pl.ds(off, _L)]
            d1 = dbufs[r][pl.ds(off + 1, _L)]
            c0 = cbufs[r][pl.ds(off, _L)]
            c1 = cbufs[r][pl.ds(off + 1, _L)]
            gidx = off + iota
            disc = (jnp.abs(d0 - d1) > _THR) & (gidx < _X - 1)
            score = jnp.where(disc, (c0 + c1) * 0.5, inf)
            scand, sidx = plsc.sort_key_val(score, gidx)
            mk = jnp.where(lo8, bests[r], lax.rev(scand, (0,)))
            mi = jnp.where(lo8, bidxs[r], lax.rev(sidx, (0,)))
            b2, i2 = plsc.sort_key_val(mk, mi)
            nb.append(b2)
            ni.append(i2)
            nc.append(cnts[r] + disc.astype(jnp.int32))
        return (tuple(nb), tuple(ni), tuple(nc))

    carry = (tuple(jnp.full((_L,), inf) for _ in range(_RPW)),
             tuple(jnp.zeros((_L,), jnp.int32) for _ in range(_RPW)),
             tuple(jnp.zeros((_L,), jnp.int32) for _ in range(_RPW)))
    spc = _CW // _L
    for cp in copies[0]:
        cp.wait()
    for ch in range(_NCH):
        if ch + 1 < _NCH:
            for cp in copies[ch + 1]:
                cp.wait()
        carry = lax.fori_loop(ch * spc, (ch + 1) * spc, body, carry)
    bests, bidxs, cnts = carry

    for r in range(_RPW):
        bv, bi, cv = bests[r], bidxs[r], cnts[r]
        uLg = plsc.load_gather(dbufs[r], [bi])
        uRg = plsc.load_gather(dbufs[r], [bi + 1])
        cLg = plsc.load_gather(cbufs[r], [bi])
        cRg = plsc.load_gather(cbufs[r], [bi + 1])
        st[r, pl.ds(0, _L)] = uLg
        st[r, pl.ds(_L, _L)] = uRg
        st[r, pl.ds(2 * _L, _L)] = (cLg + cRg) * 0.5
        st[r, pl.ds(3 * _L, _L)] = jnp.where(bv < inf, jnp.ones((_L,), jnp.float32),
                                             jnp.zeros((_L,), jnp.float32))
        st[r, pl.ds(4 * _L, _L)] = lax.broadcast(jnp.sum(cv).astype(jnp.float32), (_L,))

    pltpu.sync_copy(st, out_hbm.at[pl.ds(base, _RPW)])


def _ln(h, g, b):
    mu = jnp.mean(h, axis=-1, keepdims=True)
    var = jnp.mean((h - mu) ** 2, axis=-1, keepdims=True)
    return (h - mu) / jnp.sqrt(var + 1e-5) * g + b


def _gelu(h):
    return 0.5 * h * (1.0 + jax.lax.erf(h * 0.7071067811865476))


def _mlp_kernel(fr_ref, Win_ref, bin_ref, gin_ref, bein_ref,
                g1_ref, be1_ref, W1a_ref, b1a_ref, W1b_ref, b1b_ref,
                g2_ref, be2_ref, W2a_ref, b2a_ref, W2b_ref, b2b_ref,
                g3_ref, be3_ref, W3a_ref, b3a_ref, W3b_ref, b3b_ref,
                gh_ref, beh_ref, Wh1_ref, bh1_ref, Wh2_ref, bh2_ref,
                o_ref):
    uL = fr_ref[:, 0:_K]
    uR = fr_ref[:, _L:_L + _K]
    diff = uL - uR
    feats = (uL, uR, diff, jnp.abs(diff), (uL + uR) * 0.5, jnp.sign(diff))
    Win = Win_ref[...]
    h3 = feats[0][:, :, None] * Win[0][None, None, :]
    for f in range(1, 6):
        h3 = h3 + feats[f][:, :, None] * Win[f][None, None, :]
    bb, kk, hh = h3.shape
    h = h3.reshape(bb * kk, hh) + bin_ref[...]
    h = _gelu(_ln(h, gin_ref[...], bein_ref[...]))
    for (g_r, be_r, Wa_r, ba_r, Wb_r, bb_r) in (
            (g1_ref, be1_ref, W1a_ref, b1a_ref, W1b_ref, b1b_ref),
            (g2_ref, be2_ref, W2a_ref, b2a_ref, W2b_ref, b2b_ref),
            (g3_ref, be3_ref, W3a_ref, b3a_ref, W3b_ref, b3b_ref)):
        r = _ln(h, g_r[...], be_r[...])
        r = _gelu(jnp.dot(r, Wa_r[...], preferred_element_type=jnp.float32) + ba_r[...])
        r = jnp.dot(r, Wb_r[...], preferred_element_type=jnp.float32) + bb_r[...]
        h = h + r
    o = _ln(h, gh_ref[...], beh_ref[...])
    o = _gelu(jnp.dot(o, Wh1_ref[...], preferred_element_type=jnp.float32) + bh1_ref[...])
    o = jnp.dot(o, Wh2_ref[...], preferred_element_type=jnp.float32) + bh2_ref[...]
    o_ref[...] = o


def kernel(x, W_in, b_in, g_in, be_in, g1, be1, W1a, b1a, W1b, b1b,
           g2, be2, W2a, b2a, W2b, b2b, g3, be3, W3a, b3a, W3b, b3b,
           g_h, be_h, Wh1, bh1, Wh2, bh2):
    B, C, X = x.shape
    f32 = jnp.float32

    front = pl.kernel(
        _front_sc,
        out_type=jax.ShapeDtypeStruct((B, 5 * _L), f32),
        mesh=plsc.VectorSubcoreMesh(core_axis_name="c", subcore_axis_name="s",
                                    num_cores=2, num_subcores=16),
        compiler_params=pltpu.CompilerParams(needs_layout_passes=False),
        scratch_types=[pltpu.VMEM((_X + _L,), f32)] * 8
        + [pltpu.VMEM((_RPW, 5 * _L), f32)]
        + [pltpu.SemaphoreType.DMA] * _NCH,
    )
    fr = front(x)

    Wh1p = jnp.pad(Wh1, ((0, 0), (0, _H - Wh1.shape[1])))
    bh1p = jnp.pad(bh1, (0, _H - bh1.shape[0]))
    Wh2p = jnp.pad(Wh2, ((0, _H - Wh2.shape[0]), (0, _H - Wh2.shape[1])))
    bh2p = jnp.pad(bh2, (0, _H - bh2.shape[0]))
    row = lambda v: v.reshape(1, -1)
    o_pad = pl.pallas_call(
        _mlp_kernel,
        out_shape=jax.ShapeDtypeStruct((B * _K, _H), f32),
    )(fr, W_in, row(b_in), row(g_in), row(be_in),
      row(g1), row(be1), W1a, row(b1a), W1b, row(b1b),
      row(g2), row(be2), W2a, row(b2a), W2b, row(b2b),
      row(g3), row(be3), W3a, row(b3a), W3b, row(b3b),
      row(g_h), row(be_h), Wh1p, row(bh1p), Wh2p, row(bh2p))

    fp_mlp = o_pad[:, :2].reshape(B, _K, 2).transpose(0, 2, 1)
    fr4 = jnp.stack([fr[:, p * _L:p * _L + _K] for p in range(4)], axis=1)
    fp = jnp.concatenate([fp_mlp, fr4], axis=1)
    front_count = fr[:, 4 * _L].astype(jnp.int32)
    return (fp, front_count)


# speedup vs baseline: 1.0383x; 1.0178x over previous
import functools

import jax
import jax.numpy as jnp
from jax import lax
from jax.experimental import pallas as pl
from jax.experimental.pallas import tpu as pltpu
from jax.experimental.pallas import tpu_sc as plsc

_H = 128
_K = 8
_THR = 1e-06
_X = 8192
_L = 16
_RPW = 4


_NCH = 2
_CW = _X // _NCH


def _front_sc(x_hbm, out_hbm,
              d0_r, d1_r, d2_r, d3_r, c0_r, c1_r, c2_r, c3_r,
              st, *sems):
    wid = lax.axis_index("s") * 2 + lax.axis_index("c")
    base = wid * _RPW
    dbufs = (d0_r, d1_r, d2_r, d3_r)
    cbufs = (c0_r, c1_r, c2_r, c3_r)
    copies = [[] for _ in range(_NCH)]
    for ch in range(_NCH):
        for r in range(_RPW):
            copies[ch].append(pltpu.async_copy(
                x_hbm.at[base + r, 0, pl.ds(ch * _CW, _CW)],
                dbufs[r].at[pl.ds(ch * _CW, _CW)], sems[ch]))
            copies[ch].append(pltpu.async_copy(
                x_hbm.at[base + r, 1, pl.ds(ch * _CW, _CW)],
                cbufs[r].at[pl.ds(ch * _CW, _CW)], sems[ch]))

    inf = jnp.float32(jnp.inf)
    iota = lax.iota(jnp.int32, _L)
    lo8 = iota < _K

    def body(j, carry):
        bests, bidxs, cnts = carry
        nb, ni, nc = [], [], []
        off = j * _L
        for r in range(_RPW):
            d0 = dbufs[r][pl.ds(off, _L)]
            d1 = dbufs[r][pl.ds(off + 1, _L)]
            c0 = cbufs[r][pl.ds(off, _L)]
            c1 = cbufs[r][pl.ds(off + 1, _L)]
            gidx = off + iota
            disc = (jnp.abs(d0 - d1) > _THR) & (gidx < _X - 1)
            score = jnp.where(disc, (c0 + c1) * 0.5, inf)
            scand, sidx = plsc.sort_key_val(score, gidx)
            mk = jnp.where(lo8, bests[r], lax.rev(scand, (0,)))
            mi = jnp.where(lo8, bidxs[r], lax.rev(sidx, (0,)))
            b2, i2 = plsc.sort_key_val(mk, mi)
            nb.append(b2)
            ni.append(i2)
            nc.append(cnts[r] + disc.astype(jnp.int32))
        return (tuple(nb), tuple(ni), tuple(nc))

    carry = (tuple(jnp.full((_L,), inf) for _ in range(_RPW)),
             tuple(jnp.zeros((_L,), jnp.int32) for _ in range(_RPW)),
             tuple(jnp.zeros((_L,), jnp.int32) for _ in range(_RPW)))
    spc = _CW // _L
    for cp in copies[0]:
        cp.wait()
    for ch in range(_NCH):
        if ch + 1 < _NCH:
            for cp in copies[ch + 1]:
                cp.wait()
        carry = lax.fori_loop(ch * spc, (ch + 1) * spc, body, carry)
    bests, bidxs, cnts = carry

    for r in range(_RPW):
        bv, bi, cv = bests[r], bidxs[r], cnts[r]
        uLg = plsc.load_gather(dbufs[r], [bi])
        uRg = plsc.load_gather(dbufs[r], [bi + 1])
        cLg = plsc.load_gather(cbufs[r], [bi])
        cRg = plsc.load_gather(cbufs[r], [bi + 1])
        st[r, pl.ds(0, _L)] = uLg
        st[r, pl.ds(_L, _L)] = uRg
        st[r, pl.ds(2 * _L, _L)] = (cLg + cRg) * 0.5
        st[r, pl.ds(3 * _L, _L)] = jnp.where(bv < inf, jnp.ones((_L,), jnp.float32),
                                             jnp.zeros((_L,), jnp.float32))
        st[r, pl.ds(4 * _L, _L)] = lax.broadcast(jnp.sum(cv).astype(jnp.float32), (_L,))

    pltpu.sync_copy(st, out_hbm.at[pl.ds(base, _RPW)])


def _ln(h, g, b):
    mu = jnp.mean(h, axis=-1, keepdims=True)
    var = jnp.mean((h - mu) ** 2, axis=-1, keepdims=True)
    return (h - mu) / jnp.sqrt(var + 1e-5) * g + b


def _gelu(h):
    return 0.5 * h * (1.0 + jax.lax.erf(h * 0.7071067811865476))


def _mlp_kernel(fr_ref, Win_ref, bin_ref, gin_ref, bein_ref,
                g1_ref, be1_ref, W1a_ref, b1a_ref, W1b_ref, b1b_ref,
                g2_ref, be2_ref, W2a_ref, b2a_ref, W2b_ref, b2b_ref,
                g3_ref, be3_ref, W3a_ref, b3a_ref, W3b_ref, b3b_ref,
                gh_ref, beh_ref, Wh1_ref, bh1_ref, Wh2_ref, bh2_ref,
                o_ref):
    uL = fr_ref[:, 0:_K]
    uR = fr_ref[:, _L:_L + _K]
    diff = uL - uR
    feats = (uL, uR, diff, jnp.abs(diff), (uL + uR) * 0.5, jnp.sign(diff))
    Win = Win_ref[...]
    h3 = feats[0][:, :, None] * Win[0][None, None, :]
    for f in range(1, 6):
        h3 = h3 + feats[f][:, :, None] * Win[f][None, None, :]
    bb, kk, hh = h3.shape
    h = h3.reshape(bb * kk, hh) + bin_ref[...]
    h = _gelu(_ln(h, gin_ref[...], bein_ref[...]))
    for (g_r, be_r, Wa_r, ba_r, Wb_r, bb_r) in (
            (g1_ref, be1_ref, W1a_ref, b1a_ref, W1b_ref, b1b_ref),
            (g2_ref, be2_ref, W2a_ref, b2a_ref, W2b_ref, b2b_ref),
            (g3_ref, be3_ref, W3a_ref, b3a_ref, W3b_ref, b3b_ref)):
        r = _ln(h, g_r[...], be_r[...])
        r = _gelu(jnp.dot(r, Wa_r[...], preferred_element_type=jnp.float32) + ba_r[...])
        r = jnp.dot(r, Wb_r[...], preferred_element_type=jnp.float32) + bb_r[...]
        h = h + r
    o = _ln(h, gh_ref[...], beh_ref[...])
    o = _gelu(jnp.dot(o, Wh1_ref[...], preferred_element_type=jnp.float32) + bh1_ref[...])
    o = jnp.dot(o, Wh2_ref[...], preferred_element_type=jnp.float32) + bh2_ref[...]
    o_ref[...] = o


def kernel(x, W_in, b_in, g_in, be_in, g1, be1, W1a, b1a, W1b, b1b,
           g2, be2, W2a, b2a, W2b, b2b, g3, be3, W3a, b3a, W3b, b3b,
           g_h, be_h, Wh1, bh1, Wh2, bh2):
    B, C, X = x.shape
    f32 = jnp.float32

    front = pl.kernel(
        _front_sc,
        out_type=jax.ShapeDtypeStruct((B, 5 * _L), f32),
        mesh=plsc.VectorSubcoreMesh(core_axis_name="c", subcore_axis_name="s",
                                    num_cores=2, num_subcores=16),
        compiler_params=pltpu.CompilerParams(needs_layout_passes=False),
        scratch_types=[pltpu.VMEM((_X + _L,), f32)] * 8
        + [pltpu.VMEM((_RPW, 5 * _L), f32)]
        + [pltpu.SemaphoreType.DMA] * _NCH,
    )
    fr = front(x)

    Wh1p = jnp.pad(Wh1, ((0, 0), (0, _H - Wh1.shape[1])))
    bh1p = jnp.pad(bh1, (0, _H - bh1.shape[0]))
    Wh2p = jnp.pad(Wh2, ((0, _H - Wh2.shape[0]), (0, _H - Wh2.shape[1])))
    bh2p = jnp.pad(bh2, (0, _H - bh2.shape[0]))
    row = lambda v: v.reshape(1, -1)
    o_pad = pl.pallas_call(
        _mlp_kernel,
        out_shape=jax.ShapeDtypeStruct((B * _K, _H), f32),
    )(fr, W_in, row(b_in), row(g_in), row(be_in),
      row(g1), row(be1), W1a, row(b1a), W1b, row(b1b),
      row(g2), row(be2), W2a, row(b2a), W2b, row(b2b),
      row(g3), row(be3), W3a, row(b3a), W3b, row(b3b),
      row(g_h), row(be_h), Wh1p, row(bh1p), Wh2p, row(bh2p))

    fp_mlp = o_pad[:, :2].reshape(B, _K, 2).transpose(0, 2, 1)
    fr4 = jnp.stack([fr[:, p * _L:p * _L + _K] for p in range(4)], axis=1)
    fp = jnp.concatenate([fp_mlp, fr4], axis=1)
    front_count = fr[:, 4 * _L].astype(jnp.int32)
    return (fp, front_count)


# speedup vs baseline: 1.0522x; 1.0135x over previous
import functools

import jax
import jax.numpy as jnp
from jax import lax
from jax.experimental import pallas as pl
from jax.experimental.pallas import tpu as pltpu
from jax.experimental.pallas import tpu_sc as plsc

_H = 128
_K = 8
_THR = 1e-06
_X = 8192
_L = 16
_RPW = 4


def _front_sc(x_hbm, out_hbm,
              d0_r, d1_r, d2_r, d3_r, c0_r, c1_r, c2_r, c3_r,
              st, sem):
    wid = lax.axis_index("s") * 2 + lax.axis_index("c")
    base = wid * _RPW
    dbufs = (d0_r, d1_r, d2_r, d3_r)
    cbufs = (c0_r, c1_r, c2_r, c3_r)
    copies = []
    for r in range(_RPW):
        copies.append(pltpu.async_copy(x_hbm.at[base + r, 0], dbufs[r].at[pl.ds(0, _X)], sem))
        copies.append(pltpu.async_copy(x_hbm.at[base + r, 1], cbufs[r].at[pl.ds(0, _X)], sem))
    for cp in copies:
        cp.wait()

    inf = jnp.float32(jnp.inf)
    iota = lax.iota(jnp.int32, _L)
    lo8 = iota < _K

    def body(j, carry):
        bests, bidxs, cnts = carry
        nb, ni, nc = [], [], []
        off = j * _L
        for r in range(_RPW):
            d0 = dbufs[r][pl.ds(off, _L)]
            d1 = dbufs[r][pl.ds(off + 1, _L)]
            c0 = cbufs[r][pl.ds(off, _L)]
            c1 = cbufs[r][pl.ds(off + 1, _L)]
            gidx = off + iota
            disc = (jnp.abs(d0 - d1) > _THR) & (gidx < _X - 1)
            score = jnp.where(disc, (c0 + c1) * 0.5, inf)
            scand, sidx = plsc.sort_key_val(score, gidx)
            mk = jnp.where(lo8, bests[r], lax.rev(scand, (0,)))
            mi = jnp.where(lo8, bidxs[r], lax.rev(sidx, (0,)))
            b2, i2 = plsc.sort_key_val(mk, mi)
            nb.append(b2)
            ni.append(i2)
            nc.append(cnts[r] + disc.astype(jnp.int32))
        return (tuple(nb), tuple(ni), tuple(nc))

    init = (tuple(jnp.full((_L,), inf) for _ in range(_RPW)),
            tuple(jnp.zeros((_L,), jnp.int32) for _ in range(_RPW)),
            tuple(jnp.zeros((_L,), jnp.int32) for _ in range(_RPW)))
    bests, bidxs, cnts = lax.fori_loop(0, _X // _L, body, init)

    for r in range(_RPW):
        bv, bi, cv = bests[r], bidxs[r], cnts[r]
        uLg = plsc.load_gather(dbufs[r], [bi])
        uRg = plsc.load_gather(dbufs[r], [bi + 1])
        cLg = plsc.load_gather(cbufs[r], [bi])
        cRg = plsc.load_gather(cbufs[r], [bi + 1])
        st[r, pl.ds(0, _L)] = uLg
        st[r, pl.ds(_L, _L)] = uRg
        st[r, pl.ds(2 * _L, _L)] = (cLg + cRg) * 0.5
        st[r, pl.ds(3 * _L, _L)] = jnp.where(bv < inf, jnp.ones((_L,), jnp.float32),
                                             jnp.zeros((_L,), jnp.float32))
        st[r, pl.ds(4 * _L, _L)] = lax.broadcast(jnp.sum(cv).astype(jnp.float32), (_L,))

    pltpu.sync_copy(st, out_hbm.at[pl.ds(base, _RPW)])


def _ln(h, g, b):
    mu = jnp.mean(h, axis=-1, keepdims=True)
    var = jnp.mean((h - mu) ** 2, axis=-1, keepdims=True)
    return (h - mu) / jnp.sqrt(var + 1e-5) * g + b


def _gelu(h):
    return 0.5 * h * (1.0 + jax.lax.erf(h * 0.7071067811865476))


def _mlp_kernel(fr_ref, Win_ref, bin_ref, gin_ref, bein_ref,
                g1_ref, be1_ref, W1a_ref, b1a_ref, W1b_ref, b1b_ref,
                g2_ref, be2_ref, W2a_ref, b2a_ref, W2b_ref, b2b_ref,
                g3_ref, be3_ref, W3a_ref, b3a_ref, W3b_ref, b3b_ref,
                gh_ref, beh_ref, Wh1_ref, bh1_ref, Wh2_ref, bh2_ref,
                o_ref):
    uL = fr_ref[:, 0:_K]
    uR = fr_ref[:, _L:_L + _K]
    diff = uL - uR
    feats = (uL, uR, diff, jnp.abs(diff), (uL + uR) * 0.5, jnp.sign(diff))
    Win = Win_ref[...]
    h3 = feats[0][:, :, None] * Win[0][None, None, :]
    for f in range(1, 6):
        h3 = h3 + feats[f][:, :, None] * Win[f][None, None, :]
    bb, kk, hh = h3.shape
    h = h3.reshape(bb * kk, hh) + bin_ref[...]
    h = _gelu(_ln(h, gin_ref[...], bein_ref[...]))
    for (g_r, be_r, Wa_r, ba_r, Wb_r, bb_r) in (
            (g1_ref, be1_ref, W1a_ref, b1a_ref, W1b_ref, b1b_ref),
            (g2_ref, be2_ref, W2a_ref, b2a_ref, W2b_ref, b2b_ref),
            (g3_ref, be3_ref, W3a_ref, b3a_ref, W3b_ref, b3b_ref)):
        r = _ln(h, g_r[...], be_r[...])
        r = _gelu(jnp.dot(r, Wa_r[...], preferred_element_type=jnp.float32) + ba_r[...])
        r = jnp.dot(r, Wb_r[...], preferred_element_type=jnp.float32) + bb_r[...]
        h = h + r
    o = _ln(h, gh_ref[...], beh_ref[...])
    o = _gelu(jnp.dot(o, Wh1_ref[...], preferred_element_type=jnp.float32) + bh1_ref[...])
    o = jnp.dot(o, Wh2_ref[...], preferred_element_type=jnp.float32) + bh2_ref[...]
    o_ref[...] = o


def kernel(x, W_in, b_in, g_in, be_in, g1, be1, W1a, b1a, W1b, b1b,
           g2, be2, W2a, b2a, W2b, b2b, g3, be3, W3a, b3a, W3b, b3b,
           g_h, be_h, Wh1, bh1, Wh2, bh2):
    B, C, X = x.shape
    f32 = jnp.float32

    front = pl.kernel(
        _front_sc,
        out_type=jax.ShapeDtypeStruct((B, 5 * _L), f32),
        mesh=plsc.VectorSubcoreMesh(core_axis_name="c", subcore_axis_name="s",
                                    num_cores=2, num_subcores=16),
        compiler_params=pltpu.CompilerParams(
            needs_layout_passes=False,
            skip_device_barrier=True,
            disable_bounds_checks=True,
            disable_semaphore_checks=True,
        ),
        scratch_types=[pltpu.VMEM((_X + _L,), f32)] * 8
        + [pltpu.VMEM((_RPW, 5 * _L), f32), pltpu.SemaphoreType.DMA],
    )
    fr = front(x)

    Wh1p = jnp.pad(Wh1, ((0, 0), (0, _H - Wh1.shape[1])))
    bh1p = jnp.pad(bh1, (0, _H - bh1.shape[0]))
    Wh2p = jnp.pad(Wh2, ((0, _H - Wh2.shape[0]), (0, _H - Wh2.shape[1])))
    bh2p = jnp.pad(bh2, (0, _H - bh2.shape[0]))
    row = lambda v: v.reshape(1, -1)
    o_pad = pl.pallas_call(
        _mlp_kernel,
        out_shape=jax.ShapeDtypeStruct((B * _K, _H), f32),
        compiler_params=pltpu.CompilerParams(skip_device_barrier=True,
                                             disable_bounds_checks=True),
    )(fr, W_in, row(b_in), row(g_in), row(be_in),
      row(g1), row(be1), W1a, row(b1a), W1b, row(b1b),
      row(g2), row(be2), W2a, row(b2a), W2b, row(b2b),
      row(g3), row(be3), W3a, row(b3a), W3b, row(b3b),
      row(g_h), row(be_h), Wh1p, row(bh1p), Wh2p, row(bh2p))

    fp_mlp = o_pad[:, :2].reshape(B, _K, 2).transpose(0, 2, 1)
    fr4 = jnp.stack([fr[:, p * _L:p * _L + _K] for p in range(4)], axis=1)
    fp = jnp.concatenate([fp_mlp, fr4], axis=1)
    front_count = fr[:, 4 * _L].astype(jnp.int32)
    return (fp, front_count)


# speedup vs baseline: 1.1032x; 1.0485x over previous
import functools

import jax
import jax.numpy as jnp
from jax import lax
from jax.experimental import pallas as pl
from jax.experimental.pallas import tpu as pltpu
from jax.experimental.pallas import tpu_sc as plsc

_H = 128
_K = 8
_THR = 1e-06
_X = 8192
_L = 16
_RPW = 4


def _front_sc(x_hbm, out_hbm,
              d0_r, d1_r, d2_r, d3_r, c0_r, c1_r, c2_r, c3_r,
              st, sem):
    wid = lax.axis_index("s") * 2 + lax.axis_index("c")
    base = wid * _RPW
    dbufs = (d0_r, d1_r, d2_r, d3_r)
    cbufs = (c0_r, c1_r, c2_r, c3_r)
    copies = []
    for r in range(_RPW):
        copies.append(pltpu.async_copy(x_hbm.at[base + r, 0], dbufs[r].at[pl.ds(0, _X)], sem))
        copies.append(pltpu.async_copy(x_hbm.at[base + r, 1], cbufs[r].at[pl.ds(0, _X)], sem))
    for cp in copies:
        cp.wait()

    inf = jnp.float32(jnp.inf)
    iota = lax.iota(jnp.int32, _L)
    lo8 = iota < _K

    def body(j, carry):
        bests, bidxs, cnts = carry
        nb, ni, nc = [], [], []
        off = j * _L
        for r in range(_RPW):
            d0 = dbufs[r][pl.ds(off, _L)]
            d1 = dbufs[r][pl.ds(off + 1, _L)]
            c0 = cbufs[r][pl.ds(off, _L)]
            c1 = cbufs[r][pl.ds(off + 1, _L)]
            gidx = off + iota
            disc = (jnp.abs(d0 - d1) > _THR) & (gidx < _X - 1)
            score = jnp.where(disc, (c0 + c1) * 0.5, inf)
            scand, sidx = plsc.sort_key_val(score, gidx)
            mk = jnp.where(lo8, bests[r], lax.rev(scand, (0,)))
            mi = jnp.where(lo8, bidxs[r], lax.rev(sidx, (0,)))
            b2, i2 = plsc.sort_key_val(mk, mi)
            nb.append(b2)
            ni.append(i2)
            nc.append(cnts[r] + disc.astype(jnp.int32))
        return (tuple(nb), tuple(ni), tuple(nc))

    init = (tuple(jnp.full((_L,), inf) for _ in range(_RPW)),
            tuple(jnp.zeros((_L,), jnp.int32) for _ in range(_RPW)),
            tuple(jnp.zeros((_L,), jnp.int32) for _ in range(_RPW)))
    bests, bidxs, cnts = lax.fori_loop(0, _X // _L, body, init)

    for r in range(_RPW):
        bv, bi, cv = bests[r], bidxs[r], cnts[r]
        uLg = plsc.load_gather(dbufs[r], [bi])
        uRg = plsc.load_gather(dbufs[r], [bi + 1])
        cLg = plsc.load_gather(cbufs[r], [bi])
        cRg = plsc.load_gather(cbufs[r], [bi + 1])
        st[r, pl.ds(0, _L)] = uLg
        st[r, pl.ds(_L, _L)] = uRg
        st[r, pl.ds(2 * _L, _L)] = (cLg + cRg) * 0.5
        st[r, pl.ds(3 * _L, _L)] = jnp.where(bv < inf, jnp.ones((_L,), jnp.float32),
                                             jnp.zeros((_L,), jnp.float32))
        st[r, pl.ds(4 * _L, _L)] = lax.broadcast(jnp.sum(cv).astype(jnp.float32), (_L,))

    pltpu.sync_copy(st, out_hbm.at[pl.ds(base, _RPW)])


def _ln(h, g, b):
    mu = jnp.mean(h, axis=-1, keepdims=True)
    var = jnp.mean((h - mu) ** 2, axis=-1, keepdims=True)
    return (h - mu) / jnp.sqrt(var + 1e-5) * g + b


def _gelu(h):
    return 0.5 * h * (1.0 + jax.lax.erf(h * 0.7071067811865476))


def _mlp_kernel(fr_ref, Win_ref, bin_ref, gin_ref, bein_ref,
                g1_ref, be1_ref, W1a_ref, b1a_ref, W1b_ref, b1b_ref,
                g2_ref, be2_ref, W2a_ref, b2a_ref, W2b_ref, b2b_ref,
                g3_ref, be3_ref, W3a_ref, b3a_ref, W3b_ref, b3b_ref,
                gh_ref, beh_ref, Wh1_ref, bh1_ref, Wh2_ref, bh2_ref,
                o_ref, cnt_ref):
    uL = fr_ref[:, 0:_K]
    uR = fr_ref[:, _L:_L + _K]
    bb, kk = uL.shape
    diff = uL - uR
    feats = (uL, uR, diff, jnp.abs(diff), (uL + uR) * 0.5, jnp.sign(diff))
    Win = Win_ref[...]
    h3 = feats[0][:, :, None] * Win[0][None, None, :]
    for f in range(1, 6):
        h3 = h3 + feats[f][:, :, None] * Win[f][None, None, :]
    hh = h3.shape[-1]
    h = h3.reshape(bb * kk, hh) + bin_ref[...]
    h = _gelu(_ln(h, gin_ref[...], bein_ref[...]))
    for (g_r, be_r, Wa_r, ba_r, Wb_r, bb_r) in (
            (g1_ref, be1_ref, W1a_ref, b1a_ref, W1b_ref, b1b_ref),
            (g2_ref, be2_ref, W2a_ref, b2a_ref, W2b_ref, b2b_ref),
            (g3_ref, be3_ref, W3a_ref, b3a_ref, W3b_ref, b3b_ref)):
        r = _ln(h, g_r[...], be_r[...])
        r = _gelu(jnp.dot(r, Wa_r[...], preferred_element_type=jnp.float32) + ba_r[...])
        r = jnp.dot(r, Wb_r[...], preferred_element_type=jnp.float32) + bb_r[...]
        h = h + r
    o = _ln(h, gh_ref[...], beh_ref[...])
    o = _gelu(jnp.dot(o, Wh1_ref[...], preferred_element_type=jnp.float32) + bh1_ref[...])
    o = jnp.dot(o, Wh2_ref[...], preferred_element_type=jnp.float32) + bh2_ref[...]
    for c in range(2):
        o_ref[:, pl.ds(c, 1), :] = o[:, c].reshape(bb, 1, kk)
    for p in range(4):
        o_ref[:, pl.ds(2 + p, 1), :] = fr_ref[:, p * _L:p * _L + _K].reshape(bb, 1, kk)
    cnt_ref[...] = fr_ref[:, 4 * _L:4 * _L + _K].astype(jnp.int32)


def kernel(x, W_in, b_in, g_in, be_in, g1, be1, W1a, b1a, W1b, b1b,
           g2, be2, W2a, b2a, W2b, b2b, g3, be3, W3a, b3a, W3b, b3b,
           g_h, be_h, Wh1, bh1, Wh2, bh2):
    B, C, X = x.shape
    f32 = jnp.float32

    front = pl.kernel(
        _front_sc,
        out_type=jax.ShapeDtypeStruct((B, 5 * _L), f32),
        mesh=plsc.VectorSubcoreMesh(core_axis_name="c", subcore_axis_name="s",
                                    num_cores=2, num_subcores=16),
        compiler_params=pltpu.CompilerParams(
            needs_layout_passes=False,
            skip_device_barrier=True,
            disable_bounds_checks=True,
            disable_semaphore_checks=True,
        ),
        scratch_types=[pltpu.VMEM((_X + _L,), f32)] * 8
        + [pltpu.VMEM((_RPW, 5 * _L), f32), pltpu.SemaphoreType.DMA],
    )
    fr = front(x)

    Wh1p = jnp.pad(Wh1, ((0, 0), (0, _H - Wh1.shape[1])))
    bh1p = jnp.pad(bh1, (0, _H - bh1.shape[0]))
    Wh2p = jnp.pad(Wh2, ((0, _H - Wh2.shape[0]), (0, _H - Wh2.shape[1])))
    bh2p = jnp.pad(bh2, (0, _H - bh2.shape[0]))
    row = lambda v: v.reshape(1, -1)
    fp, cnt = pl.pallas_call(
        _mlp_kernel,
        out_shape=[jax.ShapeDtypeStruct((B, 6, _K), f32),
                   jax.ShapeDtypeStruct((B, _K), jnp.int32)],
        compiler_params=pltpu.CompilerParams(skip_device_barrier=True,
                                             disable_bounds_checks=True),
    )(fr, W_in, row(b_in), row(g_in), row(be_in),
      row(g1), row(be1), W1a, row(b1a), W1b, row(b1b),
      row(g2), row(be2), W2a, row(b2a), W2b, row(b2b),
      row(g3), row(be3), W3a, row(b3a), W3b, row(b3b),
      row(g_h), row(be_h), Wh1p, row(bh1p), Wh2p, row(bh2p))

    front_count = cnt[:, 0]
    return (fp, front_count)


# speedup vs baseline: 1.1308x; 1.0250x over previous
import functools

import jax
import jax.numpy as jnp
from jax import lax
from jax.experimental import pallas as pl
from jax.experimental.pallas import tpu as pltpu
from jax.experimental.pallas import tpu_sc as plsc

_H = 128
_K = 8
_THR = 1e-06
_X = 8192
_L = 16
_RPW = 4


def _front_sc(x_hbm, out_hbm,
              d0_r, d1_r, d2_r, d3_r, c0_r, c1_r, c2_r, c3_r,
              st, sem):
    wid = lax.axis_index("s") * 2 + lax.axis_index("c")
    base = wid * _RPW
    dbufs = (d0_r, d1_r, d2_r, d3_r)
    cbufs = (c0_r, c1_r, c2_r, c3_r)
    copies = []
    for r in range(_RPW):
        copies.append(pltpu.async_copy(x_hbm.at[base + r, 0], dbufs[r].at[pl.ds(0, _X)], sem))
        copies.append(pltpu.async_copy(x_hbm.at[base + r, 1], cbufs[r].at[pl.ds(0, _X)], sem))
    for cp in copies:
        cp.wait()

    inf = jnp.float32(jnp.inf)
    iota = lax.iota(jnp.int32, _L)
    lo8 = iota < _K

    def body(j, carry):
        bests, bidxs, cnts = carry
        nb, ni, nc = [], [], []
        off = j * _L
        for r in range(_RPW):
            d0 = dbufs[r][pl.ds(off, _L)]
            d1 = dbufs[r][pl.ds(off + 1, _L)]
            c0 = cbufs[r][pl.ds(off, _L)]
            c1 = cbufs[r][pl.ds(off + 1, _L)]
            gidx = off + iota
            disc = (jnp.abs(d0 - d1) > _THR) & (gidx < _X - 1)
            score = jnp.where(disc, (c0 + c1) * 0.5, inf)
            scand, sidx = plsc.sort_key_val(score, gidx)
            mk = jnp.where(lo8, bests[r], lax.rev(scand, (0,)))
            mi = jnp.where(lo8, bidxs[r], lax.rev(sidx, (0,)))
            b2, i2 = plsc.sort_key_val(mk, mi)
            nb.append(b2)
            ni.append(i2)
            nc.append(cnts[r] + disc.astype(jnp.int32))
        return (tuple(nb), tuple(ni), tuple(nc))

    init = (tuple(jnp.full((_L,), inf) for _ in range(_RPW)),
            tuple(jnp.zeros((_L,), jnp.int32) for _ in range(_RPW)),
            tuple(jnp.zeros((_L,), jnp.int32) for _ in range(_RPW)))
    bests, bidxs, cnts = lax.fori_loop(0, _X // _L, body, init)

    for r in range(_RPW):
        bv, bi, cv = bests[r], bidxs[r], cnts[r]
        uLg = plsc.load_gather(dbufs[r], [bi])
        uRg = plsc.load_gather(dbufs[r], [bi + 1])
        cLg = plsc.load_gather(cbufs[r], [bi])
        cRg = plsc.load_gather(cbufs[r], [bi + 1])
        st[r, pl.ds(0, _L)] = uLg
        st[r, pl.ds(_L, _L)] = uRg
        st[r, pl.ds(2 * _L, _L)] = (cLg + cRg) * 0.5
        st[r, pl.ds(3 * _L, _L)] = jnp.where(bv < inf, jnp.ones((_L,), jnp.float32),
                                             jnp.zeros((_L,), jnp.float32))
        st[r, pl.ds(4 * _L, _L)] = lax.broadcast(jnp.sum(cv).astype(jnp.float32), (_L,))

    pltpu.sync_copy(st, out_hbm.at[pl.ds(base, _RPW)])


def _ln(h, g, b):
    mu = jnp.mean(h, axis=-1, keepdims=True)
    var = jnp.mean((h - mu) ** 2, axis=-1, keepdims=True)
    return (h - mu) / jnp.sqrt(var + 1e-5) * g + b


def _gelu(h):
    return 0.5 * h * (1.0 + jax.lax.erf(h * 0.7071067811865476))


def _mlp_kernel(fr_ref, Win_ref, bin_ref, gin_ref, bein_ref,
                g1_ref, be1_ref, W1a_ref, b1a_ref, W1b_ref, b1b_ref,
                g2_ref, be2_ref, W2a_ref, b2a_ref, W2b_ref, b2b_ref,
                g3_ref, be3_ref, W3a_ref, b3a_ref, W3b_ref, b3b_ref,
                gh_ref, beh_ref, Wh1_ref, bh1_ref, Wh2_ref, bh2_ref,
                o_ref, cnt_ref):
    uL = fr_ref[:, 0:_K]
    uR = fr_ref[:, _L:_L + _K]
    bb, kk = uL.shape
    diff = uL - uR
    feats = (uL, uR, diff, jnp.abs(diff), (uL + uR) * 0.5, jnp.sign(diff))
    Win = Win_ref[...]
    h3 = feats[0][:, :, None] * Win[0][None, None, :]
    for f in range(1, 6):
        h3 = h3 + feats[f][:, :, None] * Win[f][None, None, :]
    hh = h3.shape[-1]
    h = h3.reshape(bb * kk, hh) + bin_ref[...]
    h = _gelu(_ln(h, gin_ref[...], bein_ref[...]))
    for (g_r, be_r, Wa_r, ba_r, Wb_r, bb_r) in (
            (g1_ref, be1_ref, W1a_ref, b1a_ref, W1b_ref, b1b_ref),
            (g2_ref, be2_ref, W2a_ref, b2a_ref, W2b_ref, b2b_ref),
            (g3_ref, be3_ref, W3a_ref, b3a_ref, W3b_ref, b3b_ref)):
        r = _ln(h, g_r[...], be_r[...])
        r = _gelu(jnp.dot(r, Wa_r[...], preferred_element_type=jnp.float32) + ba_r[...])
        r = jnp.dot(r, Wb_r[...], preferred_element_type=jnp.float32) + bb_r[...]
        h = h + r
    o = _ln(h, gh_ref[...], beh_ref[...])
    o = _gelu(jnp.dot(o, Wh1_ref[...], preferred_element_type=jnp.float32) + bh1_ref[...])
    o = jnp.dot(o, Wh2_ref[...], preferred_element_type=jnp.float32) + bh2_ref[...]
    for c in range(2):
        o_ref[:, pl.ds(c * _K, _K)] = o[:, c].reshape(bb, kk)
    for p in range(4):
        o_ref[:, pl.ds((2 + p) * _K, _K)] = fr_ref[:, p * _L:p * _L + _K]
    cnt_ref[...] = fr_ref[:, 4 * _L:4 * _L + _K].astype(jnp.int32)


def kernel(x, W_in, b_in, g_in, be_in, g1, be1, W1a, b1a, W1b, b1b,
           g2, be2, W2a, b2a, W2b, b2b, g3, be3, W3a, b3a, W3b, b3b,
           g_h, be_h, Wh1, bh1, Wh2, bh2):
    B, C, X = x.shape
    f32 = jnp.float32

    front = pl.kernel(
        _front_sc,
        out_type=jax.ShapeDtypeStruct((B, 5 * _L), f32),
        mesh=plsc.VectorSubcoreMesh(core_axis_name="c", subcore_axis_name="s",
                                    num_cores=2, num_subcores=16),
        compiler_params=pltpu.CompilerParams(
            needs_layout_passes=False,
            skip_device_barrier=True,
            disable_bounds_checks=True,
            disable_semaphore_checks=True,
        ),
        scratch_types=[pltpu.VMEM((_X + _L,), f32)] * 8
        + [pltpu.VMEM((_RPW, 5 * _L), f32), pltpu.SemaphoreType.DMA],
    )
    fr = front(x)

    Wh1p = jnp.pad(Wh1, ((0, 0), (0, _H - Wh1.shape[1])))
    bh1p = jnp.pad(bh1, (0, _H - bh1.shape[0]))
    Wh2p = jnp.pad(Wh2, ((0, _H - Wh2.shape[0]), (0, _H - Wh2.shape[1])))
    bh2p = jnp.pad(bh2, (0, _H - bh2.shape[0]))
    row = lambda v: v.reshape(1, -1)
    fp48, cnt = pl.pallas_call(
        _mlp_kernel,
        out_shape=[jax.ShapeDtypeStruct((B, 6 * _K), f32),
                   jax.ShapeDtypeStruct((B, _K), jnp.int32)],
        compiler_params=pltpu.CompilerParams(skip_device_barrier=True,
                                             disable_bounds_checks=True),
    )(fr, W_in, row(b_in), row(g_in), row(be_in),
      row(g1), row(be1), W1a, row(b1a), W1b, row(b1b),
      row(g2), row(be2), W2a, row(b2a), W2b, row(b2b),
      row(g3), row(be3), W3a, row(b3a), W3b, row(b3b),
      row(g_h), row(be_h), Wh1p, row(bh1p), Wh2p, row(bh2p))

    fp = fp48.reshape(B, 6, _K)
    front_count = cnt[:, 0]
    return (fp, front_count)


# speedup vs baseline: 1.1561x; 1.0224x over previous
import functools

import jax
import jax.numpy as jnp
from jax import lax
from jax.experimental import pallas as pl
from jax.experimental.pallas import tpu as pltpu
from jax.experimental.pallas import tpu_sc as plsc

_H = 128
_K = 8
_THR = 1e-06
_X = 8192
_L = 16
_RPW = 4


def _front_sc(x_hbm, out_hbm,
              d0_r, d1_r, d2_r, d3_r, c0_r, c1_r, c2_r, c3_r,
              st, sem):
    wid = lax.axis_index("s") * 2 + lax.axis_index("c")
    base = wid * _RPW
    dbufs = (d0_r, d1_r, d2_r, d3_r)
    cbufs = (c0_r, c1_r, c2_r, c3_r)
    copies = []
    for r in range(_RPW):
        copies.append(pltpu.async_copy(x_hbm.at[base + r, 0], dbufs[r].at[pl.ds(0, _X)], sem))
        copies.append(pltpu.async_copy(x_hbm.at[base + r, 1], cbufs[r].at[pl.ds(0, _X)], sem))
    for cp in copies:
        cp.wait()

    inf = jnp.float32(jnp.inf)
    iota = lax.iota(jnp.int32, _L)
    lo8 = iota < _K

    def step(off, carry, tail):
        bests, bidxs, cnts = carry
        nb, ni, nc = [], [], []
        for r in range(_RPW):
            d0 = dbufs[r][pl.ds(off, _L)]
            d1 = dbufs[r][pl.ds(off + 1, _L)]
            c0 = cbufs[r][pl.ds(off, _L)]
            c1 = cbufs[r][pl.ds(off + 1, _L)]
            gidx = off + iota
            disc = jnp.abs(d0 - d1) > _THR
            if tail:
                disc = disc & (gidx < _X - 1)
            score = jnp.where(disc, (c0 + c1) * 0.5, inf)
            scand, sidx = plsc.sort_key_val(score, gidx, descending=True)
            mk = jnp.where(lo8, bests[r], scand)
            mi = jnp.where(lo8, bidxs[r], sidx)
            b2, i2 = plsc.sort_key_val(mk, mi)
            nb.append(b2)
            ni.append(i2)
            nc.append(cnts[r] + disc.astype(jnp.int32))
        return (tuple(nb), tuple(ni), tuple(nc))

    def body(j, carry):
        return step(j * _L, carry, False)

    init = (tuple(jnp.full((_L,), inf) for _ in range(_RPW)),
            tuple(jnp.zeros((_L,), jnp.int32) for _ in range(_RPW)),
            tuple(jnp.zeros((_L,), jnp.int32) for _ in range(_RPW)))
    carry = lax.fori_loop(0, _X // _L - 1, body, init)
    bests, bidxs, cnts = step(_X - _L, carry, True)

    for r in range(_RPW):
        bv, bi, cv = bests[r], bidxs[r], cnts[r]
        uLg = plsc.load_gather(dbufs[r], [bi])
        uRg = plsc.load_gather(dbufs[r], [bi + 1])
        cLg = plsc.load_gather(cbufs[r], [bi])
        cRg = plsc.load_gather(cbufs[r], [bi + 1])
        st[r, pl.ds(0, _L)] = uLg
        st[r, pl.ds(_L, _L)] = uRg
        st[r, pl.ds(2 * _L, _L)] = (cLg + cRg) * 0.5
        st[r, pl.ds(3 * _L, _L)] = jnp.where(bv < inf, jnp.ones((_L,), jnp.float32),
                                             jnp.zeros((_L,), jnp.float32))
        st[r, pl.ds(4 * _L, _L)] = lax.broadcast(jnp.sum(cv).astype(jnp.float32), (_L,))

    pltpu.sync_copy(st, out_hbm.at[pl.ds(base, _RPW)])


def _ln(h, g, b):
    mu = jnp.mean(h, axis=-1, keepdims=True)
    var = jnp.mean((h - mu) ** 2, axis=-1, keepdims=True)
    return (h - mu) / jnp.sqrt(var + 1e-5) * g + b


def _gelu(h):
    return 0.5 * h * (1.0 + jax.lax.erf(h * 0.7071067811865476))


def _mlp_kernel(fr_ref, Win_ref, bin_ref, gin_ref, bein_ref,
                g1_ref, be1_ref, W1a_ref, b1a_ref, W1b_ref, b1b_ref,
                g2_ref, be2_ref, W2a_ref, b2a_ref, W2b_ref, b2b_ref,
                g3_ref, be3_ref, W3a_ref, b3a_ref, W3b_ref, b3b_ref,
                gh_ref, beh_ref, Wh1_ref, bh1_ref, Wh2_ref, bh2_ref,
                o_ref, cnt_ref):
    uL = fr_ref[:, 0:_K]
    uR = fr_ref[:, _L:_L + _K]
    bb, kk = uL.shape
    diff = uL - uR
    feats = (uL, uR, diff, jnp.abs(diff), (uL + uR) * 0.5, jnp.sign(diff))
    Win = Win_ref[...]
    h3 = feats[0][:, :, None] * Win[0][None, None, :]
    for f in range(1, 6):
        h3 = h3 + feats[f][:, :, None] * Win[f][None, None, :]
    hh = h3.shape[-1]
    h = h3.reshape(bb * kk, hh) + bin_ref[...]
    h = _gelu(_ln(h, gin_ref[...], bein_ref[...]))
    for (g_r, be_r, Wa_r, ba_r, Wb_r, bb_r) in (
            (g1_ref, be1_ref, W1a_ref, b1a_ref, W1b_ref, b1b_ref),
            (g2_ref, be2_ref, W2a_ref, b2a_ref, W2b_ref, b2b_ref),
            (g3_ref, be3_ref, W3a_ref, b3a_ref, W3b_ref, b3b_ref)):
        r = _ln(h, g_r[...], be_r[...])
        r = _gelu(jnp.dot(r, Wa_r[...], preferred_element_type=jnp.float32) + ba_r[...])
        r = jnp.dot(r, Wb_r[...], preferred_element_type=jnp.float32) + bb_r[...]
        h = h + r
    o = _ln(h, gh_ref[...], beh_ref[...])
    o = _gelu(jnp.dot(o, Wh1_ref[...], preferred_element_type=jnp.float32) + bh1_ref[...])
    o = jnp.dot(o, Wh2_ref[...], preferred_element_type=jnp.float32) + bh2_ref[...]
    for c in range(2):
        o_ref[:, pl.ds(c * _K, _K)] = o[:, c].reshape(bb, kk)
    for p in range(4):
        o_ref[:, pl.ds((2 + p) * _K, _K)] = fr_ref[:, p * _L:p * _L + _K]
    cnt_ref[...] = fr_ref[:, 4 * _L:4 * _L + _K].astype(jnp.int32)


def kernel(x, W_in, b_in, g_in, be_in, g1, be1, W1a, b1a, W1b, b1b,
           g2, be2, W2a, b2a, W2b, b2b, g3, be3, W3a, b3a, W3b, b3b,
           g_h, be_h, Wh1, bh1, Wh2, bh2):
    B, C, X = x.shape
    f32 = jnp.float32

    front = pl.kernel(
        _front_sc,
        out_type=jax.ShapeDtypeStruct((B, 5 * _L), f32),
        mesh=plsc.VectorSubcoreMesh(core_axis_name="c", subcore_axis_name="s",
                                    num_cores=2, num_subcores=16),
        compiler_params=pltpu.CompilerParams(
            needs_layout_passes=False,
            skip_device_barrier=True,
            disable_bounds_checks=True,
            disable_semaphore_checks=True,
        ),
        scratch_types=[pltpu.VMEM((_X + _L,), f32)] * 8
        + [pltpu.VMEM((_RPW, 5 * _L), f32), pltpu.SemaphoreType.DMA],
    )
    fr = front(x)

    Wh1p = jnp.pad(Wh1, ((0, 0), (0, _H - Wh1.shape[1])))
    bh1p = jnp.pad(bh1, (0, _H - bh1.shape[0]))
    Wh2p = jnp.pad(Wh2, ((0, _H - Wh2.shape[0]), (0, _H - Wh2.shape[1])))
    bh2p = jnp.pad(bh2, (0, _H - bh2.shape[0]))
    row = lambda v: v.reshape(1, -1)
    fp48, cnt = pl.pallas_call(
        _mlp_kernel,
        out_shape=[jax.ShapeDtypeStruct((B, 6 * _K), f32),
                   jax.ShapeDtypeStruct((B, _K), jnp.int32)],
        compiler_params=pltpu.CompilerParams(skip_device_barrier=True,
                                             disable_bounds_checks=True),
    )(fr, W_in, row(b_in), row(g_in), row(be_in),
      row(g1), row(be1), W1a, row(b1a), W1b, row(b1b),
      row(g2), row(be2), W2a, row(b2a), W2b, row(b2b),
      row(g3), row(be3), W3a, row(b3a), W3b, row(b3b),
      row(g_h), row(be_h), Wh1p, row(bh1p), Wh2p, row(bh2p))

    fp = fp48.reshape(B, 6, _K)
    front_count = cnt[:, 0]
    return (fp, front_count)


# speedup vs baseline: 1.1683x; 1.0105x over previous
import functools

import jax
import jax.numpy as jnp
from jax import lax
from jax.experimental import pallas as pl
from jax.experimental.pallas import tpu as pltpu
from jax.experimental.pallas import tpu_sc as plsc

_H = 128
_K = 8
_THR = 1e-06
_X = 8192
_L = 16
_RPW = 4


def _front_sc(x_hbm, out_hbm,
              d0_r, d1_r, d2_r, d3_r, c0_r, c1_r, c2_r, c3_r,
              st, sem):
    wid = lax.axis_index("s") * 2 + lax.axis_index("c")
    base = wid * _RPW
    dbufs = (d0_r, d1_r, d2_r, d3_r)
    cbufs = (c0_r, c1_r, c2_r, c3_r)
    copies = []
    for r in range(_RPW):
        copies.append(pltpu.async_copy(x_hbm.at[base + r, 0], dbufs[r].at[pl.ds(0, _X)], sem))
        copies.append(pltpu.async_copy(x_hbm.at[base + r, 1], cbufs[r].at[pl.ds(0, _X)], sem))
    for cp in copies:
        cp.wait()

    inf = jnp.float32(jnp.inf)
    iota = lax.iota(jnp.int32, _L)
    lo8 = iota < _K

    def step(off, carry, tail):
        bests, bidxs, cnts = carry
        nb, ni, nc = [], [], []
        for r in range(_RPW):
            d0 = dbufs[r][pl.ds(off, _L)]
            d1 = dbufs[r][pl.ds(off + 1, _L)]
            c0 = cbufs[r][pl.ds(off, _L)]
            c1 = cbufs[r][pl.ds(off + 1, _L)]
            gidx = off + iota
            disc = jnp.abs(d0 - d1) > _THR
            if tail:
                disc = disc & (gidx < _X - 1)
            score = jnp.where(disc, (c0 + c1) * 0.5, inf)
            scand, sidx = plsc.sort_key_val(score, gidx, descending=True)
            mk = jnp.where(lo8, bests[r], scand)
            mi = jnp.where(lo8, bidxs[r], sidx)
            b2, i2 = plsc.sort_key_val(mk, mi)
            nb.append(b2)
            ni.append(i2)
            nc.append(cnts[r] + disc.astype(jnp.int32))
        return (tuple(nb), tuple(ni), tuple(nc))

    def body(j, carry):
        return step(j * _L, carry, False)

    init = (tuple(jnp.full((_L,), inf) for _ in range(_RPW)),
            tuple(jnp.zeros((_L,), jnp.int32) for _ in range(_RPW)),
            tuple(jnp.zeros((_L,), jnp.int32) for _ in range(_RPW)))
    carry = lax.fori_loop(0, _X // _L - 1, body, init)
    bests, bidxs, cnts = step(_X - _L, carry, True)

    for r in range(_RPW):
        bv, bi, cv = bests[r], bidxs[r], cnts[r]
        uLg = plsc.load_gather(dbufs[r], [bi])
        uRg = plsc.load_gather(dbufs[r], [bi + 1])
        cLg = plsc.load_gather(cbufs[r], [bi])
        cRg = plsc.load_gather(cbufs[r], [bi + 1])
        st[r, pl.ds(0, _L)] = uLg
        st[r, pl.ds(_L, _L)] = uRg
        st[r, pl.ds(2 * _L, _L)] = (cLg + cRg) * 0.5
        st[r, pl.ds(3 * _L, _L)] = jnp.where(bv < inf, jnp.ones((_L,), jnp.float32),
                                             jnp.zeros((_L,), jnp.float32))
        st[r, pl.ds(4 * _L, _L)] = lax.broadcast(jnp.sum(cv).astype(jnp.float32), (_L,))

    pltpu.sync_copy(st, out_hbm.at[pl.ds(base, _RPW)])


def _ln(h, g, b, M):
    mu = jnp.dot(h, M, preferred_element_type=jnp.float32)
    e = h - mu
    var = jnp.dot(e * e, M, preferred_element_type=jnp.float32)
    return e / jnp.sqrt(var + 1e-5) * g + b


def _gelu(h):
    return 0.5 * h * (1.0 + jax.lax.erf(h * 0.7071067811865476))


def _mlp_kernel(fr_ref, Win_ref, bin_ref, gin_ref, bein_ref,
                g1_ref, be1_ref, W1a_ref, b1a_ref, W1b_ref, b1b_ref,
                g2_ref, be2_ref, W2a_ref, b2a_ref, W2b_ref, b2b_ref,
                g3_ref, be3_ref, W3a_ref, b3a_ref, W3b_ref, b3b_ref,
                gh_ref, beh_ref, Wh1_ref, bh1_ref, Wh2_ref, bh2_ref,
                o_ref, cnt_ref):
    uL = fr_ref[:, 0:_K]
    uR = fr_ref[:, _L:_L + _K]
    bb, kk = uL.shape
    diff = uL - uR
    feats = (uL, uR, diff, jnp.abs(diff), (uL + uR) * 0.5, jnp.sign(diff))
    Win = Win_ref[...]
    h3 = feats[0][:, :, None] * Win[0][None, None, :]
    for f in range(1, 6):
        h3 = h3 + feats[f][:, :, None] * Win[f][None, None, :]
    hh = h3.shape[-1]
    M = jnp.full((hh, hh), 1.0 / hh, jnp.float32)
    h = h3.reshape(bb * kk, hh) + bin_ref[...]
    h = _gelu(_ln(h, gin_ref[...], bein_ref[...], M))
    for (g_r, be_r, Wa_r, ba_r, Wb_r, bb_r) in (
            (g1_ref, be1_ref, W1a_ref, b1a_ref, W1b_ref, b1b_ref),
            (g2_ref, be2_ref, W2a_ref, b2a_ref, W2b_ref, b2b_ref),
            (g3_ref, be3_ref, W3a_ref, b3a_ref, W3b_ref, b3b_ref)):
        r = _ln(h, g_r[...], be_r[...], M)
        r = _gelu(jnp.dot(r, Wa_r[...], preferred_element_type=jnp.float32) + ba_r[...])
        r = jnp.dot(r, Wb_r[...], preferred_element_type=jnp.float32) + bb_r[...]
        h = h + r
    o = _ln(h, gh_ref[...], beh_ref[...], M)
    o = _gelu(jnp.dot(o, Wh1_ref[...], preferred_element_type=jnp.float32) + bh1_ref[...])
    o = jnp.dot(o, Wh2_ref[...], preferred_element_type=jnp.float32) + bh2_ref[...]
    for c in range(2):
        o_ref[:, pl.ds(c * _K, _K)] = o[:, c].reshape(bb, kk)
    for p in range(4):
        o_ref[:, pl.ds((2 + p) * _K, _K)] = fr_ref[:, p * _L:p * _L + _K]
    cnt_ref[...] = fr_ref[:, 4 * _L:4 * _L + _K].astype(jnp.int32)


def kernel(x, W_in, b_in, g_in, be_in, g1, be1, W1a, b1a, W1b, b1b,
           g2, be2, W2a, b2a, W2b, b2b, g3, be3, W3a, b3a, W3b, b3b,
           g_h, be_h, Wh1, bh1, Wh2, bh2):
    B, C, X = x.shape
    f32 = jnp.float32

    front = pl.kernel(
        _front_sc,
        out_type=jax.ShapeDtypeStruct((B, 5 * _L), f32),
        mesh=plsc.VectorSubcoreMesh(core_axis_name="c", subcore_axis_name="s",
                                    num_cores=2, num_subcores=16),
        compiler_params=pltpu.CompilerParams(
            needs_layout_passes=False,
            skip_device_barrier=True,
            disable_bounds_checks=True,
            disable_semaphore_checks=True,
        ),
        scratch_types=[pltpu.VMEM((_X + _L,), f32)] * 8
        + [pltpu.VMEM((_RPW, 5 * _L), f32), pltpu.SemaphoreType.DMA],
    )
    fr = front(x)

    Wh1p = jnp.pad(Wh1, ((0, 0), (0, _H - Wh1.shape[1])))
    bh1p = jnp.pad(bh1, (0, _H - bh1.shape[0]))
    Wh2p = jnp.pad(Wh2, ((0, _H - Wh2.shape[0]), (0, _H - Wh2.shape[1])))
    bh2p = jnp.pad(bh2, (0, _H - bh2.shape[0]))
    row = lambda v: v.reshape(1, -1)
    fp48, cnt = pl.pallas_call(
        _mlp_kernel,
        out_shape=[jax.ShapeDtypeStruct((B, 6 * _K), f32),
                   jax.ShapeDtypeStruct((B, _K), jnp.int32)],
        compiler_params=pltpu.CompilerParams(skip_device_barrier=True,
                                             disable_bounds_checks=True),
    )(fr, W_in, row(b_in), row(g_in), row(be_in),
      row(g1), row(be1), W1a, row(b1a), W1b, row(b1b),
      row(g2), row(be2), W2a, row(b2a), W2b, row(b2b),
      row(g3), row(be3), W3a, row(b3a), W3b, row(b3b),
      row(g_h), row(be_h), Wh1p, row(bh1p), Wh2p, row(bh2p))

    fp = fp48.reshape(B, 6, _K)
    front_count = cnt[:, 0]
    return (fp, front_count)
